# baseline (speedup 1.0000x reference)
import functools

import jax
import jax.numpy as jnp
from jax.experimental import pallas as pl
from jax.experimental.pallas import tpu as pltpu


def _swa_kernel(
    x_ref,
    wq_ref,
    bq_ref,
    wk_ref,
    bk_ref,
    wv_ref,
    bv_ref,
    wr_ref,
    br_ref,
    gamma_ref,
    seg_ref,
    feat_ref,
    *,
    n_tok,
):
    d0, d1, d2, C = x_ref.shape
    rows = d0 * d1 * d2
    n_win = rows // n_tok
    xf = x_ref[...].reshape(rows, C)
    cdt = jnp.bfloat16
    xc = xf.astype(cdt)
    K = wq_ref.shape[1]
    f32 = jnp.float32

    wqkv = jnp.concatenate(
        [wq_ref[...].astype(cdt), wk_ref[...].astype(cdt),
         wv_ref[...].astype(cdt)], axis=1)
    wrc = wr_ref[...].astype(cdt)

    qkv = jnp.dot(xc, wqkv, preferred_element_type=f32)
    q = qkv[:, :K] + bq_ref[...]
    k = qkv[:, K:2 * K] + bk_ref[...]
    v = qkv[:, 2 * K:] + bv_ref[...]

    seg_ref[...] = q.reshape(seg_ref.shape)

    q3 = q.astype(cdt).reshape(n_win, n_tok, K)
    k3 = k.astype(cdt).reshape(n_win, n_tok, K)
    v3 = v.astype(cdt).reshape(n_win, n_tok, C)

    s = jnp.einsum("bnk,bmk->bnm", q3, k3, preferred_element_type=f32)
    s = s - jnp.max(s, axis=-1, keepdims=True)
    p = jnp.exp(s)
    recip = pl.reciprocal(jnp.sum(p, axis=-1, keepdims=True), approx=True)

    o = jnp.einsum("bnm,bmc->bnc", p.astype(cdt), v3, preferred_element_type=f32)

    gamma = gamma_ref[0]
    o2 = o.reshape(rows, C).astype(cdt)
    r = jnp.dot(o2, wrc, preferred_element_type=f32)
    scale = (gamma * recip).reshape(rows, 1)
    feat_ref[...] = (scale * r + (gamma * br_ref[...] + xf)).reshape(
        feat_ref.shape)


def kernel(x, wq, bq, wk, bk, wv, bv, wr, br, gamma):
    B, H, W, C = x.shape
    ws = 8
    K = wq.shape[1]

    HB = 8 * ws
    while H % HB:
        HB //= 2
    grid = (H // HB, W // ws)

    wmap = lambda i, j: (0, 0)
    xmap = lambda i, j: (0, i, j, 0)

    seg, feat = pl.pallas_call(
        functools.partial(_swa_kernel, n_tok=ws * ws),
        out_shape=(
            jax.ShapeDtypeStruct((B, H, W, K), jnp.float32),
            jax.ShapeDtypeStruct((B, H, W, C), jnp.float32),
        ),
        grid=grid,
        in_specs=[
            pl.BlockSpec((B, HB, ws, C), xmap),
            pl.BlockSpec((C, K), wmap),
            pl.BlockSpec((1, K), wmap),
            pl.BlockSpec((C, K), wmap),
            pl.BlockSpec((1, K), wmap),
            pl.BlockSpec((C, C), wmap),
            pl.BlockSpec((1, C), wmap),
            pl.BlockSpec((C, C), wmap),
            pl.BlockSpec((1, C), wmap),
            pl.BlockSpec(memory_space=pltpu.MemorySpace.SMEM),
        ],
        out_specs=[
            pl.BlockSpec((B, HB, ws, K), xmap),
            pl.BlockSpec((B, HB, ws, C), xmap),
        ],
        compiler_params=pltpu.CompilerParams(
            dimension_semantics=("parallel", "parallel")),
    )(x, wq, bq, wk, bk, wv, bv, wr, br, gamma)

    return seg, feat


# speedup vs baseline: 1.0073x; 1.0073x over previous
import functools

import jax
import jax.numpy as jnp
from jax.experimental import pallas as pl
from jax.experimental.pallas import tpu as pltpu


def _swa_kernel(
    x_ref,
    wq_ref,
    bq_ref,
    wk_ref,
    bk_ref,
    wv_ref,
    bv_ref,
    wr_ref,
    br_ref,
    gamma_ref,
    seg_ref,
    feat_ref,
    *,
    n_tok,
):
    d0, d1, d2, C = x_ref.shape
    rows = d0 * d1 * d2
    n_win = rows // n_tok
    xf = x_ref[...].reshape(rows, C)
    cdt = jnp.bfloat16
    xc = xf.astype(cdt)
    K = wq_ref.shape[1]
    f32 = jnp.float32

    wqkv = jnp.concatenate(
        [wq_ref[...].astype(cdt), wk_ref[...].astype(cdt),
         wv_ref[...].astype(cdt)], axis=1)
    wrc = wr_ref[...].astype(cdt)

    qkv = jnp.dot(xc, wqkv, preferred_element_type=f32)
    q = qkv[:, :K] + bq_ref[...]
    k = qkv[:, K:2 * K] + bk_ref[...]
    v = qkv[:, 2 * K:] + bv_ref[...]

    seg_ref[...] = q.reshape(seg_ref.shape)

    q3 = q.astype(cdt).reshape(n_win, n_tok, K)
    k3 = k.astype(cdt).reshape(n_win, n_tok, K)
    v3 = v.astype(cdt).reshape(n_win, n_tok, C)

    s = jnp.einsum("bnk,bmk->bnm", q3, k3, preferred_element_type=f32)
    s = s - jnp.max(s, axis=-1, keepdims=True)
    p = jnp.exp(s)
    p = p * pl.reciprocal(jnp.sum(p, axis=-1, keepdims=True), approx=True)

    o = jnp.einsum("bnm,bmc->bnc", p.astype(cdt), v3, preferred_element_type=f32)

    o2 = o.reshape(rows, C).astype(cdt)
    r = jnp.dot(o2, wrc, preferred_element_type=f32) + br_ref[...]
    feat_ref[...] = (gamma_ref[0] * r + xf).reshape(feat_ref.shape)


def kernel(x, wq, bq, wk, bk, wv, bv, wr, br, gamma):
    B, H, W, C = x.shape
    ws = 8
    K = wq.shape[1]

    HB = 8 * ws
    while H % HB:
        HB //= 2
    grid = (H // HB, W // ws)

    wmap = lambda i, j: (0, 0)
    xmap = lambda i, j: (0, i, j, 0)

    seg, feat = pl.pallas_call(
        functools.partial(_swa_kernel, n_tok=ws * ws),
        out_shape=(
            jax.ShapeDtypeStruct((B, H, W, K), jnp.float32),
            jax.ShapeDtypeStruct((B, H, W, C), jnp.float32),
        ),
        grid=grid,
        in_specs=[
            pl.BlockSpec((B, HB, ws, C), xmap),
            pl.BlockSpec((C, K), wmap),
            pl.BlockSpec((1, K), wmap),
            pl.BlockSpec((C, K), wmap),
            pl.BlockSpec((1, K), wmap),
            pl.BlockSpec((C, C), wmap),
            pl.BlockSpec((1, C), wmap),
            pl.BlockSpec((C, C), wmap),
            pl.BlockSpec((1, C), wmap),
            pl.BlockSpec(memory_space=pltpu.MemorySpace.SMEM),
        ],
        out_specs=[
            pl.BlockSpec((B, HB, ws, K), xmap),
            pl.BlockSpec((B, HB, ws, C), xmap),
        ],
        compiler_params=pltpu.CompilerParams(
            dimension_semantics=("parallel", "parallel")),
    )(x, wq, bq, wk, bk, wv, bv, wr, br, gamma)

    return seg, feat
